# baseline (speedup 1.0000x reference)
import math

import jax
import jax.numpy as jnp
from jax import lax
from jax.experimental import pallas as pl
from jax.experimental.pallas import tpu as pltpu
from jax.experimental.pallas import tpu_sc as plsc

VOCAB = 1000000
D = 64
SCALE = math.sqrt(D)
LANES = 16

NC = 2
NS = 16
NW = NC * NS

S = 4096
P = 200
SB = S // 128
UNITS = P * SB
U_PER_W = UNITS // NW


def _body(tok_hbm, table_hbm, out_hbm, idx_v, rows_v, yblk_v, sem_g, sem_o):
    wid = lax.axis_index("s") * NC + lax.axis_index("c")
    u0 = wid * U_PER_W

    iota = lax.iota(jnp.int32, LANES)
    row_sel = [iota + LANES * j for j in range(128 // LANES)]

    def unit_pq(u):
        return u // SB, lax.rem(u, SB)

    def idx_load(u, b):
        p, sb = unit_pq(u)
        pltpu.sync_copy(tok_hbm.at[p, pl.ds(sb * 128, 128)], idx_v.at[b])

    def gather_start(u, b):
        pltpu.make_async_copy(
            table_hbm.at[idx_v.at[b]], rows_v.at[b], sem_g.at[b]
        ).start()

    def gather_wait(b):
        pltpu.make_async_copy(
            table_hbm.at[idx_v.at[b]], rows_v.at[b], sem_g.at[b]
        ).wait()

    def out_start(u, b):
        p, sb = unit_pq(u)
        for a in range(8):
            pltpu.make_async_copy(
                yblk_v.at[b, pl.ds(8 * a, 8)], out_hbm.at[p, a, sb], sem_o.at[b]
            ).start()

    def out_wait(u, b):
        p, sb = unit_pq(u)
        for a in range(8):
            pltpu.make_async_copy(
                yblk_v.at[b, pl.ds(8 * a, 8)], out_hbm.at[p, a, sb], sem_o.at[b]
            ).wait()

    idx_load(u0, 0)
    gather_start(u0, 0)

    def outer(uo, _):
        for b in range(2):
            u = u0 + uo * 2 + b

            @pl.when(uo * 2 + b >= 2)
            def _():
                out_wait(u - 2, b)

            @pl.when(uo * 2 + b + 1 < U_PER_W)
            def _():
                idx_load(u + 1, 1 - b)
                gather_start(u + 1, 1 - b)

            gather_wait(b)

            rows = rows_v.at[b]
            yblk = yblk_v.at[b]

            @plsc.parallel_loop(0, D, step=1, unroll=2)
            def _(r):
                col = jnp.broadcast_to(r, (LANES,)).astype(jnp.int32)
                for j in range(128 // LANES):
                    vals = plsc.load_gather(rows, [row_sel[j], col])
                    yblk[r, pl.ds(LANES * j, LANES)] = vals * SCALE

            out_start(u, b)
        return 0

    lax.fori_loop(0, U_PER_W // 2, outer, 0)

    out_wait(u0 + U_PER_W - 2, 0)
    out_wait(u0 + U_PER_W - 1, 1)


def kernel(tokens, embedding):
    tok_t = tokens.astype(jnp.int32).T
    emb_lin = lax.optimization_barrier(
        embedding.reshape(VOCAB // 2, 2 * D)
    ).reshape(VOCAB, D)
    mesh = plsc.VectorSubcoreMesh(core_axis_name="c", subcore_axis_name="s")
    out5 = pl.kernel(
        _body,
        out_type=jax.ShapeDtypeStruct((P, 8, SB, 8, 128), jnp.float32),
        mesh=mesh,
        scratch_types=[
            pltpu.VMEM((2, 128), jnp.int32),
            pltpu.VMEM((2, 128, D), jnp.float32),
            pltpu.VMEM((2, D, 128), jnp.float32),
            pltpu.SemaphoreType.DMA((2,)),
            pltpu.SemaphoreType.DMA((2,)),
        ],
        compiler_params=pltpu.CompilerParams(
            use_tc_tiling_on_sc=False, needs_layout_passes=False
        ),
    )(tok_t, emb_lin)
    return out5.transpose(2, 4, 0, 1, 3).reshape(S, P, D)
